# Initial kernel scaffold; baseline (speedup 1.0000x reference)
import jax
import jax.numpy as jnp
from jax.experimental import pallas as pl


def kernel(x, edge_index, W0, b0, W1, b1, W2, b2, W3, b3, Wp1, bp1, Wp2, bp2):
    raise NotImplementedError("write your pallas kernel here")


# speedup vs baseline: 18.4025x; 18.4025x over previous
import functools

import jax
import jax.numpy as jnp
from jax import lax
from jax.experimental import pallas as pl
from jax.experimental.pallas import tpu as pltpu
from jax.experimental.pallas import tpu_sc as plsc

N = 10000
E = 320000
DH = 128
DOUT = 64

NC = 2
NS = 16
NW = NC * NS
ET = E // NW
CH = 128
NCHP = 80
ETP = CH * NCHP
PAD = ETP - ET
NPAD = 10240
ROWS_T = NPAD // NS
ZR = 40

_mesh = plsc.VectorSubcoreMesh(core_axis_name="c", subcore_axis_name="s")


@functools.partial(
    pl.kernel,
    mesh=_mesh,
    out_type=(
        jax.ShapeDtypeStruct((NPAD,), jnp.float32),
        jax.ShapeDtypeStruct((NPAD,), jnp.float32),
    ),
    scratch_types=[
        pltpu.VMEM((NCHP, CH), jnp.int32),
        pltpu.VMEM((CH,), jnp.float32),
        pltpu.VMEM((ROWS_T,), jnp.float32),
        pltpu.VMEM_SHARED((NPAD,), jnp.float32),
    ],
)
def _hist(dst_hbm, out0, out1, didx, ones, zbuf, deg_sp):
    c = lax.axis_index("c")
    s = lax.axis_index("s")
    w = c * NS + s
    pltpu.sync_copy(dst_hbm.at[w], didx)
    for k in range(CH // 16):
        ones[pl.ds(k * 16, 16)] = jnp.ones((16,), jnp.float32)
    for k in range(ROWS_T // 16):
        zbuf[pl.ds(k * 16, 16)] = jnp.zeros((16,), jnp.float32)
    pltpu.sync_copy(zbuf, deg_sp.at[pl.ds(s * ROWS_T, ROWS_T)])
    plsc.subcore_barrier()

    def body(j, _):
        pltpu.sync_copy(ones, deg_sp.at[didx.at[j]], add=True)
        return 0
    lax.fori_loop(0, NCHP, body, 0)
    plsc.subcore_barrier()

    @pl.when(c == 0)
    def _():
        pltpu.sync_copy(deg_sp.at[pl.ds(s * ROWS_T, ROWS_T)],
                        out0.at[pl.ds(s * ROWS_T, ROWS_T)])

    @pl.when(c == 1)
    def _():
        pltpu.sync_copy(deg_sp.at[pl.ds(s * ROWS_T, ROWS_T)],
                        out1.at[pl.ds(s * ROWS_T, ROWS_T)])


@functools.partial(
    pl.kernel,
    mesh=_mesh,
    out_type=(
        jax.ShapeDtypeStruct((NPAD, DH), jnp.float32),
        jax.ShapeDtypeStruct((NPAD, DH), jnp.float32),
    ),
    scratch_types=[
        pltpu.VMEM((2, CH), jnp.int32),
        pltpu.VMEM((2, CH), jnp.int32),
        pltpu.VMEM((CH, DH), jnp.float32),
        pltpu.VMEM((CH, DH), jnp.float32),
        pltpu.VMEM((ZR, DH), jnp.float32),
        pltpu.SemaphoreType.DMA,
        pltpu.SemaphoreType.DMA,
        pltpu.SemaphoreType.DMA,
        pltpu.SemaphoreType.DMA,
        pltpu.VMEM_SHARED((NPAD, DH), jnp.float32),
    ],
)
def _agg(y_hbm, src_hbm, dst_hbm, out0, out1,
         sidx, didx, buf0, buf1, zbuf, gsem0, gsem1, isem0, isem1, t_sp):
    c = lax.axis_index("c")
    s = lax.axis_index("s")
    w = c * NS + s

    bufs = (buf0, buf1)
    gsems = (gsem0, gsem1)
    isems = (isem0, isem1)

    def load_idx(j, slot):
        pltpu.async_copy(src_hbm.at[w, j], sidx.at[slot], isems[slot])
        pltpu.async_copy(dst_hbm.at[w, j], didx.at[slot], isems[slot])

    def wait_idx(slot):
        pltpu.make_async_copy(src_hbm.at[w, 0], sidx.at[slot], isems[slot]).wait()
        pltpu.make_async_copy(dst_hbm.at[w, 0], didx.at[slot], isems[slot]).wait()

    load_idx(0, 0)
    load_idx(1, 1)

    def zrow(i, _):
        for k in range(DH // 16):
            zbuf[i, pl.ds(k * 16, 16)] = jnp.zeros((16,), jnp.float32)
        return 0
    lax.fori_loop(0, ZR, zrow, 0)
    row0 = s * ROWS_T
    for q in range(ROWS_T // ZR):
        pltpu.sync_copy(zbuf, t_sp.at[pl.ds(row0 + q * ZR, ZR)])
    plsc.subcore_barrier()

    wait_idx(0)
    pltpu.async_copy(y_hbm.at[sidx.at[0]], buf0, gsem0)

    def step(j, b, nb, do_load):
        wait_idx(nb)
        pltpu.async_copy(y_hbm.at[sidx.at[nb]], bufs[nb], gsems[nb])
        pltpu.make_async_copy(y_hbm.at[sidx.at[b]], bufs[b], gsems[b]).wait()
        pltpu.sync_copy(bufs[b], t_sp.at[didx.at[b]], add=True)
        if do_load:
            load_idx(j + 2, b)

    def body(jo, _):
        step(2 * jo, 0, 1, True)
        step(2 * jo + 1, 1, 0, True)
        return 0
    lax.fori_loop(0, (NCHP - 2) // 2, body, 0)

    wait_idx(1)
    pltpu.async_copy(y_hbm.at[sidx.at[1]], buf1, gsem1)
    pltpu.make_async_copy(y_hbm.at[sidx.at[0]], buf0, gsem0).wait()
    pltpu.sync_copy(buf0, t_sp.at[didx.at[0]], add=True)
    pltpu.make_async_copy(y_hbm.at[sidx.at[1]], buf1, gsem1).wait()
    pltpu.sync_copy(buf1, t_sp.at[didx.at[1]], add=True)

    plsc.subcore_barrier()

    @pl.when(c == 0)
    def _():
        pltpu.sync_copy(t_sp.at[pl.ds(row0, ROWS_T)],
                        out0.at[pl.ds(row0, ROWS_T)])

    @pl.when(c == 1)
    def _():
        pltpu.sync_copy(t_sp.at[pl.ds(row0, ROWS_T)],
                        out1.at[pl.ds(row0, ROWS_T)])


BR = 1000


def _tc_call(body, out_blocks, in_specs, out_specs):
    return pl.pallas_call(
        body,
        grid=(N // BR,),
        in_specs=in_specs,
        out_specs=out_specs,
        out_shape=out_blocks,
    )


_row = pl.BlockSpec((BR, DH), lambda i: (i, 0))
_col = pl.BlockSpec((BR, 1), lambda i: (i, 0))
_wfull = pl.BlockSpec((DH, DH), lambda i: (0, 0))
_bfull = pl.BlockSpec((1, DH), lambda i: (0, 0))


def _prologue_body(x_ref, w_ref, da_ref, db_ref, y_ref, dinv_ref):
    deg = da_ref[...] + db_ref[...] + 1.0
    dinv = lax.rsqrt(deg)
    h = jnp.dot(x_ref[...], w_ref[...], preferred_element_type=jnp.float32)
    y_ref[...] = dinv * h
    dinv_ref[...] = dinv


_prologue = _tc_call(
    _prologue_body,
    (jax.ShapeDtypeStruct((N, DH), jnp.float32),
     jax.ShapeDtypeStruct((N, 1), jnp.float32)),
    [_row, _wfull, _col, _col],
    (_row, _col),
)


def _fuse_body(t0_ref, t1_ref, y_ref, dinv_ref, w_ref, b_ref, out_ref):
    dinv = dinv_ref[...]
    z = jax.nn.relu(dinv * (t0_ref[...] + t1_ref[...] + y_ref[...]) + b_ref[...])
    out_ref[...] = dinv * jnp.dot(z, w_ref[...], preferred_element_type=jnp.float32)


_fuse = _tc_call(
    _fuse_body,
    jax.ShapeDtypeStruct((N, DH), jnp.float32),
    [_row, _row, _row, _col, _wfull, _bfull],
    _row,
)


def _head_body(t0_ref, t1_ref, y_ref, dinv_ref, b3_ref,
               wp1_ref, bp1_ref, wp2_ref, bp2_ref, out_ref):
    dinv = dinv_ref[...]
    z = jax.nn.relu(dinv * (t0_ref[...] + t1_ref[...] + y_ref[...]) + b3_ref[...])
    h = jnp.dot(z, wp1_ref[...], preferred_element_type=jnp.float32) + bp1_ref[...]
    o = jnp.dot(h, wp2_ref[...], preferred_element_type=jnp.float32) + bp2_ref[...]
    m = jnp.max(o, axis=1, keepdims=True)
    lse = jnp.log(jnp.sum(jnp.exp(o - m), axis=1, keepdims=True)) + m
    out_ref[...] = o - lse


_head = _tc_call(
    _head_body,
    jax.ShapeDtypeStruct((N, DOUT), jnp.float32),
    [_row, _row, _row, _col, _bfull,
     _wfull, pl.BlockSpec((1, DH), lambda i: (0, 0)),
     pl.BlockSpec((DH, DOUT), lambda i: (0, 0)),
     pl.BlockSpec((1, DOUT), lambda i: (0, 0))],
    pl.BlockSpec((BR, DOUT), lambda i: (i, 0)),
)


def _pad_edges(edge_index):
    src = edge_index[0].reshape(NW, ET)
    dst = edge_index[1].reshape(NW, ET)
    pad_src = jnp.broadcast_to(jnp.arange(PAD, dtype=jnp.int32), (NW, PAD))
    pad_dst = jnp.broadcast_to(N + jnp.arange(PAD, dtype=jnp.int32), (NW, PAD))
    src = jnp.concatenate([src, pad_src], axis=1).reshape(NW, NCHP, CH)
    dst = jnp.concatenate([dst, pad_dst], axis=1).reshape(NW, NCHP, CH)
    return src, dst


def kernel(x, edge_index, W0, b0, W1, b1, W2, b2, W3, b3, Wp1, bp1, Wp2, bp2):
    src, dst = _pad_edges(edge_index)

    deg0, deg1 = _hist(dst)
    da = deg0[:N].reshape(N, 1)
    db = deg1[:N].reshape(N, 1)

    y, dinv = _prologue(x, W0, da, db)
    for (W, b) in ((W1, b0), (W2, b1), (W3, b2)):
        t0, t1 = _agg(y, src, dst)
        y = _fuse(t0, t1, y, dinv, W, b.reshape(1, DH))
    t0, t1 = _agg(y, src, dst)
    return _head(t0, t1, y, dinv, b3.reshape(1, DH),
                 Wp1, bp1.reshape(1, DH), Wp2, bp2.reshape(1, DOUT))


# speedup vs baseline: 18.7525x; 1.0190x over previous
import functools

import jax
import jax.numpy as jnp
from jax import lax
from jax.experimental import pallas as pl
from jax.experimental.pallas import tpu as pltpu
from jax.experimental.pallas import tpu_sc as plsc

N = 10000
E = 320000
DH = 128
DOUT = 64

NC = 2
NS = 16
NW = NC * NS
ET = E // NW
CH = 128
NCHP = 80
ETP = CH * NCHP
PAD = ETP - ET
NPAD = 10240
ROWS_T = NPAD // NS
ZR = 40

_mesh = plsc.VectorSubcoreMesh(core_axis_name="c", subcore_axis_name="s")


@functools.partial(
    pl.kernel,
    mesh=_mesh,
    out_type=(
        jax.ShapeDtypeStruct((NPAD,), jnp.float32),
        jax.ShapeDtypeStruct((NPAD,), jnp.float32),
    ),
    scratch_types=[
        pltpu.VMEM((NCHP, CH), jnp.int32),
        pltpu.VMEM((CH,), jnp.float32),
        pltpu.VMEM((ROWS_T,), jnp.float32),
        pltpu.VMEM_SHARED((NPAD,), jnp.float32),
    ],
)
def _hist(dst_hbm, out0, out1, didx, ones, zbuf, deg_sp):
    c = lax.axis_index("c")
    s = lax.axis_index("s")
    w = c * NS + s
    pltpu.sync_copy(dst_hbm.at[w], didx)
    for k in range(CH // 16):
        ones[pl.ds(k * 16, 16)] = jnp.ones((16,), jnp.float32)
    for k in range(ROWS_T // 16):
        zbuf[pl.ds(k * 16, 16)] = jnp.zeros((16,), jnp.float32)
    pltpu.sync_copy(zbuf, deg_sp.at[pl.ds(s * ROWS_T, ROWS_T)])
    plsc.subcore_barrier()

    def body(j, _):
        pltpu.sync_copy(ones, deg_sp.at[didx.at[j]], add=True)
        return 0
    lax.fori_loop(0, NCHP - 1, body, 0)
    plsc.subcore_barrier()

    @pl.when(c == 0)
    def _():
        pltpu.sync_copy(deg_sp.at[pl.ds(s * ROWS_T, ROWS_T)],
                        out0.at[pl.ds(s * ROWS_T, ROWS_T)])

    @pl.when(c == 1)
    def _():
        pltpu.sync_copy(deg_sp.at[pl.ds(s * ROWS_T, ROWS_T)],
                        out1.at[pl.ds(s * ROWS_T, ROWS_T)])


@functools.partial(
    pl.kernel,
    mesh=_mesh,
    out_type=(
        jax.ShapeDtypeStruct((NPAD, DH), jnp.float32),
        jax.ShapeDtypeStruct((NPAD, DH), jnp.float32),
    ),
    scratch_types=[
        pltpu.VMEM((2, CH), jnp.int32),
        pltpu.VMEM((2, CH), jnp.int32),
        pltpu.VMEM((CH, DH), jnp.float32),
        pltpu.VMEM((CH, DH), jnp.float32),
        pltpu.VMEM((ZR, DH), jnp.float32),
        pltpu.SemaphoreType.DMA,
        pltpu.SemaphoreType.DMA,
        pltpu.SemaphoreType.DMA,
        pltpu.SemaphoreType.DMA,
        pltpu.VMEM_SHARED((NPAD, DH), jnp.float32),
    ],
)
def _agg(y_hbm, src_hbm, dst_hbm, out0, out1,
         sidx, didx, buf0, buf1, zbuf, gsem0, gsem1, isem0, isem1, t_sp):
    c = lax.axis_index("c")
    s = lax.axis_index("s")
    w = c * NS + s

    bufs = (buf0, buf1)
    gsems = (gsem0, gsem1)
    isems = (isem0, isem1)

    def load_idx(j, slot):
        pltpu.async_copy(src_hbm.at[w, j], sidx.at[slot], isems[slot])
        pltpu.async_copy(dst_hbm.at[w, j], didx.at[slot], isems[slot])

    def wait_idx(slot):
        pltpu.make_async_copy(src_hbm.at[w, 0], sidx.at[slot], isems[slot]).wait()
        pltpu.make_async_copy(dst_hbm.at[w, 0], didx.at[slot], isems[slot]).wait()

    load_idx(0, 0)
    load_idx(1, 1)
    wait_idx(0)
    pltpu.async_copy(y_hbm.at[sidx.at[0]], buf0, gsem0)

    def zrow(i, _):
        for k in range(DH // 16):
            zbuf[i, pl.ds(k * 16, 16)] = jnp.zeros((16,), jnp.float32)
        return 0
    lax.fori_loop(0, ZR, zrow, 0)
    row0 = s * ROWS_T
    for q in range(ROWS_T // ZR):
        pltpu.sync_copy(zbuf, t_sp.at[pl.ds(row0 + q * ZR, ZR)])
    plsc.subcore_barrier()

    def step(j, b, nb, do_gather, do_load):
        if do_gather:
            wait_idx(nb)
            pltpu.async_copy(y_hbm.at[sidx.at[nb]], bufs[nb], gsems[nb])
        pltpu.make_async_copy(y_hbm.at[sidx.at[b]], bufs[b], gsems[b]).wait()
        pltpu.sync_copy(bufs[b], t_sp.at[didx.at[b]], add=True)
        if do_load:
            load_idx(j + 2, b)

    def body(jo, _):
        step(2 * jo, 0, 1, True, True)
        step(2 * jo + 1, 1, 0, True, True)
        return 0
    lax.fori_loop(0, (NCHP - 4) // 2, body, 0)
    step(NCHP - 4, 0, 1, True, True)
    step(NCHP - 3, 1, 0, True, False)
    step(NCHP - 2, 0, 1, False, False)

    plsc.subcore_barrier()

    @pl.when(c == 0)
    def _():
        pltpu.sync_copy(t_sp.at[pl.ds(row0, ROWS_T)],
                        out0.at[pl.ds(row0, ROWS_T)])

    @pl.when(c == 1)
    def _():
        pltpu.sync_copy(t_sp.at[pl.ds(row0, ROWS_T)],
                        out1.at[pl.ds(row0, ROWS_T)])


BR = 1000


def _tc_call(body, out_blocks, in_specs, out_specs):
    return pl.pallas_call(
        body,
        grid=(N // BR,),
        in_specs=in_specs,
        out_specs=out_specs,
        out_shape=out_blocks,
    )


_row = pl.BlockSpec((BR, DH), lambda i: (i, 0))
_col = pl.BlockSpec((BR, 1), lambda i: (i, 0))
_wfull = pl.BlockSpec((DH, DH), lambda i: (0, 0))
_bfull = pl.BlockSpec((1, DH), lambda i: (0, 0))


def _prologue_body(x_ref, w_ref, da_ref, db_ref, y_ref, dinv_ref):
    deg = da_ref[...] + db_ref[...] + 1.0
    dinv = lax.rsqrt(deg)
    h = jnp.dot(x_ref[...], w_ref[...], preferred_element_type=jnp.float32)
    y_ref[...] = dinv * h
    dinv_ref[...] = dinv


_prologue = _tc_call(
    _prologue_body,
    (jax.ShapeDtypeStruct((N, DH), jnp.float32),
     jax.ShapeDtypeStruct((N, 1), jnp.float32)),
    [_row, _wfull, _col, _col],
    (_row, _col),
)


def _fuse_body(t0_ref, t1_ref, y_ref, dinv_ref, w_ref, b_ref, out_ref):
    dinv = dinv_ref[...]
    z = jax.nn.relu(dinv * (t0_ref[...] + t1_ref[...] + y_ref[...]) + b_ref[...])
    out_ref[...] = dinv * jnp.dot(z, w_ref[...], preferred_element_type=jnp.float32)


_fuse = _tc_call(
    _fuse_body,
    jax.ShapeDtypeStruct((N, DH), jnp.float32),
    [_row, _row, _row, _col, _wfull, _bfull],
    _row,
)


def _head_body(t0_ref, t1_ref, y_ref, dinv_ref, b3_ref,
               wp1_ref, bp1_ref, wp2_ref, bp2_ref, out_ref):
    dinv = dinv_ref[...]
    z = jax.nn.relu(dinv * (t0_ref[...] + t1_ref[...] + y_ref[...]) + b3_ref[...])
    h = jnp.dot(z, wp1_ref[...], preferred_element_type=jnp.float32) + bp1_ref[...]
    o = jnp.dot(h, wp2_ref[...], preferred_element_type=jnp.float32) + bp2_ref[...]
    m = jnp.max(o, axis=1, keepdims=True)
    lse = jnp.log(jnp.sum(jnp.exp(o - m), axis=1, keepdims=True)) + m
    out_ref[...] = o - lse


_head = _tc_call(
    _head_body,
    jax.ShapeDtypeStruct((N, DOUT), jnp.float32),
    [_row, _row, _row, _col, _bfull,
     _wfull, pl.BlockSpec((1, DH), lambda i: (0, 0)),
     pl.BlockSpec((DH, DOUT), lambda i: (0, 0)),
     pl.BlockSpec((1, DOUT), lambda i: (0, 0))],
    pl.BlockSpec((BR, DOUT), lambda i: (i, 0)),
)


def _pad_edges(edge_index):
    src = edge_index[0].reshape(NW, ET)
    dst = edge_index[1].reshape(NW, ET)
    pad_src = jnp.broadcast_to(jnp.arange(PAD, dtype=jnp.int32), (NW, PAD))
    pad_dst = jnp.broadcast_to(N + jnp.arange(PAD, dtype=jnp.int32), (NW, PAD))
    src = jnp.concatenate([src, pad_src], axis=1).reshape(NW, NCHP, CH)
    dst = jnp.concatenate([dst, pad_dst], axis=1).reshape(NW, NCHP, CH)
    return src, dst


def kernel(x, edge_index, W0, b0, W1, b1, W2, b2, W3, b3, Wp1, bp1, Wp2, bp2):
    src, dst = _pad_edges(edge_index)

    deg0, deg1 = _hist(dst)
    da = deg0[:N].reshape(N, 1)
    db = deg1[:N].reshape(N, 1)

    y, dinv = _prologue(x, W0, da, db)
    for (W, b) in ((W1, b0), (W2, b1), (W3, b2)):
        t0, t1 = _agg(y, src, dst)
        y = _fuse(t0, t1, y, dinv, W, b.reshape(1, DH))
    t0, t1 = _agg(y, src, dst)
    return _head(t0, t1, y, dinv, b3.reshape(1, DH),
                 Wp1, bp1.reshape(1, DH), Wp2, bp2.reshape(1, DOUT))


# speedup vs baseline: 20.9224x; 1.1157x over previous
import functools

import jax
import jax.numpy as jnp
from jax import lax
from jax.experimental import pallas as pl
from jax.experimental.pallas import tpu as pltpu
from jax.experimental.pallas import tpu_sc as plsc

N = 10000
E = 320000
DH = 128
DOUT = 64

NC = 2
NS = 16
NW = NC * NS
ET = E // NW
CH = 128
NCHP = 80
ETP = CH * NCHP
PAD = ETP - ET
NPAD = 10240
ROWS_T = NPAD // NS
ZR = 40

_mesh = plsc.VectorSubcoreMesh(core_axis_name="c", subcore_axis_name="s")


@functools.partial(
    pl.kernel,
    mesh=_mesh,
    out_type=(
        jax.ShapeDtypeStruct((NPAD,), jnp.float32),
        jax.ShapeDtypeStruct((NPAD,), jnp.float32),
    ),
    scratch_types=[
        pltpu.VMEM((NCHP, CH), jnp.int32),
        pltpu.VMEM((CH,), jnp.float32),
        pltpu.VMEM((ROWS_T,), jnp.float32),
        pltpu.VMEM_SHARED((NPAD,), jnp.float32),
    ],
)
def _hist(dst_hbm, out0, out1, didx, ones, zbuf, deg_sp):
    c = lax.axis_index("c")
    s = lax.axis_index("s")
    w = c * NS + s
    pltpu.sync_copy(dst_hbm.at[w], didx)
    for k in range(CH // 16):
        ones[pl.ds(k * 16, 16)] = jnp.ones((16,), jnp.float32)
    for k in range(ROWS_T // 16):
        zbuf[pl.ds(k * 16, 16)] = jnp.zeros((16,), jnp.float32)
    pltpu.sync_copy(zbuf, deg_sp.at[pl.ds(s * ROWS_T, ROWS_T)])
    plsc.subcore_barrier()

    def body(j, _):
        pltpu.sync_copy(ones, deg_sp.at[didx.at[j]], add=True)
        return 0
    lax.fori_loop(0, NCHP - 1, body, 0)
    plsc.subcore_barrier()

    @pl.when(c == 0)
    def _():
        pltpu.sync_copy(deg_sp.at[pl.ds(s * ROWS_T, ROWS_T)],
                        out0.at[pl.ds(s * ROWS_T, ROWS_T)])

    @pl.when(c == 1)
    def _():
        pltpu.sync_copy(deg_sp.at[pl.ds(s * ROWS_T, ROWS_T)],
                        out1.at[pl.ds(s * ROWS_T, ROWS_T)])


@functools.partial(
    pl.kernel,
    mesh=_mesh,
    out_type=(
        jax.ShapeDtypeStruct((NPAD, DH), jnp.float32),
        jax.ShapeDtypeStruct((NPAD, DH), jnp.float32),
    ),
    scratch_types=[
        pltpu.VMEM((4, CH), jnp.int32),
        pltpu.VMEM((4, CH), jnp.int32),
        pltpu.VMEM((CH, DH), jnp.float32),
        pltpu.VMEM((CH, DH), jnp.float32),
        pltpu.VMEM((ZR, DH), jnp.float32),
        pltpu.SemaphoreType.DMA,
        pltpu.SemaphoreType.DMA,
        pltpu.SemaphoreType.DMA,
        pltpu.SemaphoreType.DMA,
        pltpu.SemaphoreType.DMA,
        pltpu.SemaphoreType.DMA,
        pltpu.SemaphoreType.DMA,
        pltpu.SemaphoreType.DMA,
        pltpu.VMEM_SHARED((NPAD, DH), jnp.float32),
    ],
)
def _agg(y_hbm, src_hbm, dst_hbm, out0, out1,
         sidx, didx, buf0, buf1, zbuf, gsem0, gsem1, ssem0, ssem1,
         isem0, isem1, isem2, isem3, t_sp):
    c = lax.axis_index("c")
    s = lax.axis_index("s")
    w = c * NS + s

    bufs = (buf0, buf1)
    gsems = (gsem0, gsem1)
    ssems = (ssem0, ssem1)
    isems = (isem0, isem1, isem2, isem3)

    def load_idx(j, q):
        pltpu.async_copy(src_hbm.at[w, j], sidx.at[q], isems[q])
        pltpu.async_copy(dst_hbm.at[w, j], didx.at[q], isems[q])

    def wait_idx(q):
        pltpu.make_async_copy(src_hbm.at[w, 0], sidx.at[q], isems[q]).wait()
        pltpu.make_async_copy(dst_hbm.at[w, 0], didx.at[q], isems[q]).wait()

    def start_gather(q, b):
        pltpu.async_copy(y_hbm.at[sidx.at[q]], bufs[b], gsems[b])

    def wait_gather(b):
        pltpu.make_async_copy(y_hbm.at[sidx.at[0]], bufs[b], gsems[b]).wait()

    def start_scatter(q, b):
        pltpu.async_copy(bufs[b], t_sp.at[didx.at[q]], ssems[b], add=True)

    def wait_scatter(q, b):
        pltpu.make_async_copy(bufs[b], t_sp.at[didx.at[q]], ssems[b]).wait()

    load_idx(0, 0)
    load_idx(1, 1)
    wait_idx(0)
    start_gather(0, 0)

    def zrow(i, _):
        for k in range(DH // 16):
            zbuf[i, pl.ds(k * 16, 16)] = jnp.zeros((16,), jnp.float32)
        return 0
    lax.fori_loop(0, ZR, zrow, 0)
    row0 = s * ROWS_T
    for q in range(ROWS_T // ZR):
        pltpu.sync_copy(zbuf, t_sp.at[pl.ds(row0 + q * ZR, ZR)])
    plsc.subcore_barrier()

    wait_idx(1)
    start_gather(1, 1)
    wait_gather(0)
    start_scatter(0, 0)
    load_idx(2, 2)

    def body(jo, _):
        for u in range(4):
            jj = 1 + 4 * jo + u
            b, nb, q = (1 + u) % 2, u % 2, (1 + u) % 4
            wait_scatter(u % 4, nb)
            wait_idx((2 + u) % 4)
            start_gather((2 + u) % 4, nb)
            wait_gather(b)
            start_scatter(q, b)
            load_idx(jj + 2, (3 + u) % 4)
        return 0
    lax.fori_loop(0, 19, body, 0)
    wait_scatter(0, 0)
    wait_idx(2)
    start_gather(2, 0)
    wait_gather(1)
    start_scatter(1, 1)
    wait_scatter(1, 1)
    wait_gather(0)
    start_scatter(2, 0)
    wait_scatter(2, 0)

    plsc.subcore_barrier()

    @pl.when(c == 0)
    def _():
        pltpu.sync_copy(t_sp.at[pl.ds(row0, ROWS_T)],
                        out0.at[pl.ds(row0, ROWS_T)])

    @pl.when(c == 1)
    def _():
        pltpu.sync_copy(t_sp.at[pl.ds(row0, ROWS_T)],
                        out1.at[pl.ds(row0, ROWS_T)])


BR = 1000


def _tc_call(body, out_blocks, in_specs, out_specs):
    return pl.pallas_call(
        body,
        grid=(N // BR,),
        in_specs=in_specs,
        out_specs=out_specs,
        out_shape=out_blocks,
    )


_row = pl.BlockSpec((BR, DH), lambda i: (i, 0))
_col = pl.BlockSpec((BR, 1), lambda i: (i, 0))
_wfull = pl.BlockSpec((DH, DH), lambda i: (0, 0))
_bfull = pl.BlockSpec((1, DH), lambda i: (0, 0))


def _prologue_body(x_ref, w_ref, da_ref, db_ref, y_ref, dinv_ref):
    deg = da_ref[...] + db_ref[...] + 1.0
    dinv = lax.rsqrt(deg)
    h = jnp.dot(x_ref[...], w_ref[...], preferred_element_type=jnp.float32)
    y_ref[...] = dinv * h
    dinv_ref[...] = dinv


_prologue = _tc_call(
    _prologue_body,
    (jax.ShapeDtypeStruct((N, DH), jnp.float32),
     jax.ShapeDtypeStruct((N, 1), jnp.float32)),
    [_row, _wfull, _col, _col],
    (_row, _col),
)


def _fuse_body(t0_ref, t1_ref, y_ref, dinv_ref, w_ref, b_ref, out_ref):
    dinv = dinv_ref[...]
    z = jax.nn.relu(dinv * (t0_ref[...] + t1_ref[...] + y_ref[...]) + b_ref[...])
    out_ref[...] = dinv * jnp.dot(z, w_ref[...], preferred_element_type=jnp.float32)


_fuse = _tc_call(
    _fuse_body,
    jax.ShapeDtypeStruct((N, DH), jnp.float32),
    [_row, _row, _row, _col, _wfull, _bfull],
    _row,
)


def _head_body(t0_ref, t1_ref, y_ref, dinv_ref, b3_ref,
               wp1_ref, bp1_ref, wp2_ref, bp2_ref, out_ref):
    dinv = dinv_ref[...]
    z = jax.nn.relu(dinv * (t0_ref[...] + t1_ref[...] + y_ref[...]) + b3_ref[...])
    h = jnp.dot(z, wp1_ref[...], preferred_element_type=jnp.float32) + bp1_ref[...]
    o = jnp.dot(h, wp2_ref[...], preferred_element_type=jnp.float32) + bp2_ref[...]
    m = jnp.max(o, axis=1, keepdims=True)
    lse = jnp.log(jnp.sum(jnp.exp(o - m), axis=1, keepdims=True)) + m
    out_ref[...] = o - lse


_head = _tc_call(
    _head_body,
    jax.ShapeDtypeStruct((N, DOUT), jnp.float32),
    [_row, _row, _row, _col, _bfull,
     _wfull, pl.BlockSpec((1, DH), lambda i: (0, 0)),
     pl.BlockSpec((DH, DOUT), lambda i: (0, 0)),
     pl.BlockSpec((1, DOUT), lambda i: (0, 0))],
    pl.BlockSpec((BR, DOUT), lambda i: (i, 0)),
)


def _pad_edges(edge_index):
    src = edge_index[0].reshape(NW, ET)
    dst = edge_index[1].reshape(NW, ET)
    pad_src = jnp.broadcast_to(jnp.arange(PAD, dtype=jnp.int32), (NW, PAD))
    pad_dst = jnp.broadcast_to(N + jnp.arange(PAD, dtype=jnp.int32), (NW, PAD))
    src = jnp.concatenate([src, pad_src], axis=1).reshape(NW, NCHP, CH)
    dst = jnp.concatenate([dst, pad_dst], axis=1).reshape(NW, NCHP, CH)
    return src, dst


def kernel(x, edge_index, W0, b0, W1, b1, W2, b2, W3, b3, Wp1, bp1, Wp2, bp2):
    src, dst = _pad_edges(edge_index)

    deg0, deg1 = _hist(dst)
    da = deg0[:N].reshape(N, 1)
    db = deg1[:N].reshape(N, 1)

    y, dinv = _prologue(x, W0, da, db)
    for (W, b) in ((W1, b0), (W2, b1), (W3, b2)):
        t0, t1 = _agg(y, src, dst)
        y = _fuse(t0, t1, y, dinv, W, b.reshape(1, DH))
    t0, t1 = _agg(y, src, dst)
    return _head(t0, t1, y, dinv, b3.reshape(1, DH),
                 Wp1, bp1.reshape(1, DH), Wp2, bp2.reshape(1, DOUT))
